# baseline (speedup 1.0000x reference)
import jax
import jax.numpy as jnp
from jax.experimental import pallas as pl
from jax.experimental.pallas import tpu as pltpu

N = 512
U = 64
DIN = 2
K = 2
B = 64
M = 2 * K + 1
BB = 4


def _supports_body(adj_ref, s1_ref, s2_ref, s1sq_ref, s2sq_ref):
    adj = adj_ref[...]
    d1 = jnp.sum(adj, axis=1)
    inv1 = jnp.where(d1 > 0, 1.0 / d1, 0.0)
    s1 = (jnp.transpose(adj) * inv1[None, :]).astype(jnp.bfloat16)
    d2 = jnp.sum(adj, axis=0)
    inv2 = jnp.where(d2 > 0, 1.0 / d2, 0.0)
    s2 = (adj * inv2[None, :]).astype(jnp.bfloat16)
    s1_ref[...] = s1
    s2_ref[...] = s2
    s1sq_ref[...] = jnp.dot(
        s1, s1, preferred_element_type=jnp.float32).astype(jnp.bfloat16)
    s2sq_ref[...] = jnp.dot(
        s2, s2, preferred_element_type=jnp.float32).astype(jnp.bfloat16)


def _dcgru_body(x_ref, s1_ref, s2_ref, s1sq_ref, s2sq_ref,
                w0_ref, b0_ref, w1_ref, b1_ref, hid_ref, h2_ref):
    s1 = s1_ref[...]
    s2 = s2_ref[...]
    s1sq = s1sq_ref[...]
    s2sq = s2sq_ref[...]

    bf16 = jnp.bfloat16

    def layer(xb16, f_in, w, bias):
        z1 = jnp.dot(s1, xb16,
                     preferred_element_type=jnp.float32).astype(bf16)
        z1p = jnp.dot(s1sq, xb16,
                      preferred_element_type=jnp.float32).astype(bf16)
        z3 = jnp.dot(s2, xb16,
                     preferred_element_type=jnp.float32).astype(bf16)
        z3p = jnp.dot(s2sq, xb16,
                      preferred_element_type=jnp.float32).astype(bf16)
        hs = []
        for pr in range(BB // 2):
            sl = slice(pr * 2 * f_in, (pr + 1) * 2 * f_in)
            g = jnp.concatenate(
                [a[:, sl] for a in (xb16, z1, z1p, z3, z3p)], axis=1)
            p = jnp.dot(g, w, preferred_element_type=jnp.float32) + bias
            t = jnp.tanh(p)
            hs.append((0.5 - 0.5 * t[:, :2 * U]) * t[:, 2 * U:])
        return jnp.concatenate(hs, axis=1)

    x = jnp.concatenate([x_ref[bl].astype(bf16) for bl in range(BB)], axis=1)
    h1 = layer(x, DIN, w0_ref[...], b0_ref[...])
    for bl in range(BB):
        hid_ref[0, bl] = h1[:, bl * U:(bl + 1) * U]
    h2 = layer(h1.astype(bf16), U, w1_ref[...], b1_ref[...])
    for bl in range(BB):
        hid_ref[1, bl] = h2[:, bl * U:(bl + 1) * U]
        h2_ref[bl] = h2[:, bl * U:(bl + 1) * U]


def _prep_w(w_g, b_g, w_c, b_c, f_in):
    wg = w_g.reshape(f_in + U, M, 2 * U)[:f_in, :, U:]
    wc = w_c.reshape(f_in + U, M, U)[:f_in]
    v = jnp.concatenate([0.5 * wg, wc], axis=2)
    v0 = v[:, 0] - v[:, 2] - v[:, 4]
    terms = [v0, v[:, 1], 2.0 * v[:, 2], v[:, 3], 2.0 * v[:, 4]]
    eye = jnp.eye(2, dtype=v.dtype)
    blocks = []
    for t in terms:
        tu, tc = t[:, :U], t[:, U:]
        blocks.append(jnp.concatenate(
            [jnp.kron(eye, tu), jnp.kron(eye, tc)], axis=1))
    w = jnp.concatenate(blocks, axis=0)
    b = jnp.concatenate(
        [jnp.tile(0.5 * b_g[U:], 2), jnp.tile(b_c, 2)]).reshape(1, -1)
    return w.astype(jnp.bfloat16), b


def kernel(inputs, adj_mx, W_g0, b_g0, W_c0, b_c0, W_g1, b_g1, W_c1, b_c1):
    f32 = jnp.float32
    s1, s2, s1sq, s2sq = pl.pallas_call(
        _supports_body,
        out_shape=[jax.ShapeDtypeStruct((N, N), jnp.bfloat16)] * 4,
    )(adj_mx)

    x = inputs.reshape(B, N, DIN)
    w0, b0 = _prep_w(W_g0, b_g0, W_c0, b_c0, DIN)
    w1, b1 = _prep_w(W_g1, b_g1, W_c1, b_c1, U)

    full = lambda shape: pl.BlockSpec(shape, lambda b: (0,) * len(shape))
    hid, h2 = pl.pallas_call(
        _dcgru_body,
        grid=(B // BB,),
        in_specs=[
            pl.BlockSpec((BB, N, DIN), lambda b: (b, 0, 0)),
            full((N, N)),
            full((N, N)),
            full((N, N)),
            full((N, N)),
            full((M * 2 * DIN, 4 * U)),
            full((1, 4 * U)),
            full((M * 2 * U, 4 * U)),
            full((1, 4 * U)),
        ],
        out_specs=[
            pl.BlockSpec((2, BB, N, U), lambda b: (0, b, 0, 0)),
            pl.BlockSpec((BB, N, U), lambda b: (b, 0, 0)),
        ],
        out_shape=[jax.ShapeDtypeStruct((2, B, N, U), f32),
                   jax.ShapeDtypeStruct((B, N, U), f32)],
        compiler_params=pltpu.CompilerParams(
            dimension_semantics=("parallel",)),
    )(x, s1, s2, s1sq, s2sq, w0, b0, w1, b1)

    return (h2.reshape(B, N * U), hid.reshape(2, B, N * U))


# speedup vs baseline: 1.0916x; 1.0916x over previous
import jax
import jax.numpy as jnp
from jax.experimental import pallas as pl
from jax.experimental.pallas import tpu as pltpu

N = 512
U = 64
DIN = 2
K = 2
B = 64
M = 2 * K + 1
BB = 4


def _supports_body(adj_ref, x0_ref, s1_ref, s2_ref, s1sq_ref, s2sq_ref,
                   t_ref):
    adj = adj_ref[...]
    d1 = jnp.sum(adj, axis=1)
    inv1 = jnp.where(d1 > 0, 1.0 / d1, 0.0)
    s1 = (jnp.transpose(adj) * inv1[None, :]).astype(jnp.bfloat16)
    d2 = jnp.sum(adj, axis=0)
    inv2 = jnp.where(d2 > 0, 1.0 / d2, 0.0)
    s2 = (adj * inv2[None, :]).astype(jnp.bfloat16)
    s1_ref[...] = s1
    s2_ref[...] = s2
    s1sq = jnp.dot(
        s1, s1, preferred_element_type=jnp.float32).astype(jnp.bfloat16)
    s2sq = jnp.dot(
        s2, s2, preferred_element_type=jnp.float32).astype(jnp.bfloat16)
    s1sq_ref[...] = s1sq
    s2sq_ref[...] = s2sq
    x0 = x0_ref[...].astype(jnp.bfloat16)
    t_ref[0] = x0
    for i, s in enumerate((s1, s1sq, s2, s2sq)):
        t_ref[i + 1] = jnp.dot(
            s, x0, preferred_element_type=jnp.float32).astype(jnp.bfloat16)


def _dcgru_body(t_ref, s1_ref, s2_ref, s1sq_ref, s2sq_ref,
                w0_ref, b0_ref, w1_ref, b1_ref, hid_ref, h2_ref):
    s1 = s1_ref[...]
    s2 = s2_ref[...]
    s1sq = s1sq_ref[...]
    s2sq = s2sq_ref[...]

    bf16 = jnp.bfloat16

    def layer(xb16, f_in, w, bias):
        z1 = jnp.dot(s1, xb16,
                     preferred_element_type=jnp.float32).astype(bf16)
        z1p = jnp.dot(s1sq, xb16,
                      preferred_element_type=jnp.float32).astype(bf16)
        z3 = jnp.dot(s2, xb16,
                     preferred_element_type=jnp.float32).astype(bf16)
        z3p = jnp.dot(s2sq, xb16,
                      preferred_element_type=jnp.float32).astype(bf16)
        hs = []
        for pr in range(BB // 2):
            sl = slice(pr * 2 * f_in, (pr + 1) * 2 * f_in)
            g = jnp.concatenate(
                [a[:, sl] for a in (xb16, z1, z1p, z3, z3p)], axis=1)
            p = jnp.dot(g, w, preferred_element_type=jnp.float32) + bias
            t = jnp.tanh(p)
            hs.append((0.5 - 0.5 * t[:, :2 * U]) * t[:, 2 * U:])
        return jnp.concatenate(hs, axis=1)

    t5 = t_ref[0]
    h1s = []
    for pr in range(BB // 2):
        sl = slice(pr * 2 * DIN, (pr + 1) * 2 * DIN)
        g = jnp.concatenate([t5[m][:, sl] for m in range(M)], axis=1)
        p = jnp.dot(g, w0_ref[...],
                    preferred_element_type=jnp.float32) + b0_ref[...]
        t = jnp.tanh(p)
        h1s.append((0.5 - 0.5 * t[:, :2 * U]) * t[:, 2 * U:])
    h1 = jnp.concatenate(h1s, axis=1)
    for bl in range(BB):
        hid_ref[0, bl] = h1[:, bl * U:(bl + 1) * U]
    h2 = layer(h1.astype(bf16), U, w1_ref[...], b1_ref[...])
    for bl in range(BB):
        hid_ref[1, bl] = h2[:, bl * U:(bl + 1) * U]
        h2_ref[bl] = h2[:, bl * U:(bl + 1) * U]


def _prep_w(w_g, b_g, w_c, b_c, f_in):
    wg = w_g.reshape(f_in + U, M, 2 * U)[:f_in, :, U:]
    wc = w_c.reshape(f_in + U, M, U)[:f_in]
    v = jnp.concatenate([0.5 * wg, wc], axis=2)
    v0 = v[:, 0] - v[:, 2] - v[:, 4]
    terms = [v0, v[:, 1], 2.0 * v[:, 2], v[:, 3], 2.0 * v[:, 4]]
    eye = jnp.eye(2, dtype=v.dtype)
    blocks = []
    for t in terms:
        tu, tc = t[:, :U], t[:, U:]
        blocks.append(jnp.concatenate(
            [jnp.kron(eye, tu), jnp.kron(eye, tc)], axis=1))
    w = jnp.concatenate(blocks, axis=0)
    b = jnp.concatenate(
        [jnp.tile(0.5 * b_g[U:], 2), jnp.tile(b_c, 2)]).reshape(1, -1)
    return w.astype(jnp.bfloat16), b


def kernel(inputs, adj_mx, W_g0, b_g0, W_c0, b_c0, W_g1, b_g1, W_c1, b_c1):
    f32 = jnp.float32
    x0 = inputs.reshape(B, N, DIN).transpose(1, 0, 2).reshape(N, B * DIN)
    s1, s2, s1sq, s2sq, tms = pl.pallas_call(
        _supports_body,
        out_shape=[jax.ShapeDtypeStruct((N, N), jnp.bfloat16)] * 4
        + [jax.ShapeDtypeStruct((M, N, B * DIN), jnp.bfloat16)],
    )(adj_mx, x0)
    tms = tms.reshape(M, N, B // BB, BB * DIN).transpose(2, 0, 1, 3)
    w0, b0 = _prep_w(W_g0, b_g0, W_c0, b_c0, DIN)
    w1, b1 = _prep_w(W_g1, b_g1, W_c1, b_c1, U)

    full = lambda shape: pl.BlockSpec(shape, lambda b: (0,) * len(shape))
    hid, h2 = pl.pallas_call(
        _dcgru_body,
        grid=(B // BB,),
        in_specs=[
            pl.BlockSpec((1, M, N, BB * DIN), lambda b: (b, 0, 0, 0)),
            full((N, N)),
            full((N, N)),
            full((N, N)),
            full((N, N)),
            full((M * 2 * DIN, 4 * U)),
            full((1, 4 * U)),
            full((M * 2 * U, 4 * U)),
            full((1, 4 * U)),
        ],
        out_specs=[
            pl.BlockSpec((2, BB, N, U), lambda b: (0, b, 0, 0)),
            pl.BlockSpec((BB, N, U), lambda b: (b, 0, 0)),
        ],
        out_shape=[jax.ShapeDtypeStruct((2, B, N, U), f32),
                   jax.ShapeDtypeStruct((B, N, U), f32)],
        compiler_params=pltpu.CompilerParams(
            dimension_semantics=("parallel",)),
    )(tms, s1, s2, s1sq, s2sq, w0, b0, w1, b1)

    return (h2.reshape(B, N * U), hid.reshape(2, B, N * U))


# speedup vs baseline: 1.1764x; 1.0776x over previous
import jax
import jax.numpy as jnp
from jax.experimental import pallas as pl
from jax.experimental.pallas import tpu as pltpu

N = 512
U = 64
DIN = 2
K = 2
B = 64
M = 2 * K + 1
BB = 8


def _supports_body(adj_ref, x0_ref, s1_ref, s2_ref, s1sq_ref, s2sq_ref,
                   t_ref):
    adj = adj_ref[...]
    d1 = jnp.sum(adj, axis=1)
    inv1 = jnp.where(d1 > 0, 1.0 / d1, 0.0)
    s1 = (jnp.transpose(adj) * inv1[None, :]).astype(jnp.bfloat16)
    d2 = jnp.sum(adj, axis=0)
    inv2 = jnp.where(d2 > 0, 1.0 / d2, 0.0)
    s2 = (adj * inv2[None, :]).astype(jnp.bfloat16)
    s1_ref[...] = s1
    s2_ref[...] = s2
    s1sq = jnp.dot(
        s1, s1, preferred_element_type=jnp.float32).astype(jnp.bfloat16)
    s2sq = jnp.dot(
        s2, s2, preferred_element_type=jnp.float32).astype(jnp.bfloat16)
    s1sq_ref[...] = s1sq
    s2sq_ref[...] = s2sq
    x0 = x0_ref[...].astype(jnp.bfloat16)
    t_ref[0] = x0
    for i, s in enumerate((s1, s1sq, s2, s2sq)):
        t_ref[i + 1] = jnp.dot(
            s, x0, preferred_element_type=jnp.float32).astype(jnp.bfloat16)


def _dcgru_body(t_ref, s1_ref, s2_ref, s1sq_ref, s2sq_ref,
                w0_ref, b0_ref, w1_ref, b1_ref, hid_ref, h2_ref):
    s1 = s1_ref[...]
    s2 = s2_ref[...]
    s1sq = s1sq_ref[...]
    s2sq = s2sq_ref[...]

    bf16 = jnp.bfloat16

    def layer(xb16, f_in, w, bias):
        z1 = jnp.dot(s1, xb16,
                     preferred_element_type=jnp.float32).astype(bf16)
        z1p = jnp.dot(s1sq, xb16,
                      preferred_element_type=jnp.float32).astype(bf16)
        z3 = jnp.dot(s2, xb16,
                     preferred_element_type=jnp.float32).astype(bf16)
        z3p = jnp.dot(s2sq, xb16,
                      preferred_element_type=jnp.float32).astype(bf16)
        hs = []
        for pr in range(BB // 2):
            sl = slice(pr * 2 * f_in, (pr + 1) * 2 * f_in)
            g = jnp.concatenate(
                [a[:, sl] for a in (xb16, z1, z1p, z3, z3p)], axis=1)
            p = jnp.dot(g, w, preferred_element_type=jnp.float32) + bias
            t = jnp.tanh(p)
            hs.append((0.5 - 0.5 * t[:, :2 * U]) * t[:, 2 * U:])
        return jnp.concatenate(hs, axis=1)

    t5 = t_ref[0]
    h1s = []
    for pr in range(BB // 2):
        sl = slice(pr * 2 * DIN, (pr + 1) * 2 * DIN)
        g = jnp.concatenate([t5[m][:, sl] for m in range(M)], axis=1)
        p = jnp.dot(g, w0_ref[...],
                    preferred_element_type=jnp.float32) + b0_ref[...]
        t = jnp.tanh(p)
        h1s.append((0.5 - 0.5 * t[:, :2 * U]) * t[:, 2 * U:])
    h1 = jnp.concatenate(h1s, axis=1)
    for bl in range(BB):
        hid_ref[0, bl] = h1[:, bl * U:(bl + 1) * U]
    h2 = layer(h1.astype(bf16), U, w1_ref[...], b1_ref[...])
    for bl in range(BB):
        hid_ref[1, bl] = h2[:, bl * U:(bl + 1) * U]
        h2_ref[bl] = h2[:, bl * U:(bl + 1) * U]


def _prep_w(w_g, b_g, w_c, b_c, f_in):
    wg = w_g.reshape(f_in + U, M, 2 * U)[:f_in, :, U:]
    wc = w_c.reshape(f_in + U, M, U)[:f_in]
    v = jnp.concatenate([0.5 * wg, wc], axis=2)
    v0 = v[:, 0] - v[:, 2] - v[:, 4]
    terms = [v0, v[:, 1], 2.0 * v[:, 2], v[:, 3], 2.0 * v[:, 4]]
    eye = jnp.eye(2, dtype=v.dtype)
    blocks = []
    for t in terms:
        tu, tc = t[:, :U], t[:, U:]
        blocks.append(jnp.concatenate(
            [jnp.kron(eye, tu), jnp.kron(eye, tc)], axis=1))
    w = jnp.concatenate(blocks, axis=0)
    b = jnp.concatenate(
        [jnp.tile(0.5 * b_g[U:], 2), jnp.tile(b_c, 2)]).reshape(1, -1)
    return w.astype(jnp.bfloat16), b


def kernel(inputs, adj_mx, W_g0, b_g0, W_c0, b_c0, W_g1, b_g1, W_c1, b_c1):
    f32 = jnp.float32
    x0 = inputs.reshape(B, N, DIN).transpose(1, 0, 2).reshape(N, B * DIN)
    s1, s2, s1sq, s2sq, tms = pl.pallas_call(
        _supports_body,
        out_shape=[jax.ShapeDtypeStruct((N, N), jnp.bfloat16)] * 4
        + [jax.ShapeDtypeStruct((M, N, B * DIN), jnp.bfloat16)],
    )(adj_mx, x0)
    tms = tms.reshape(M, N, B // BB, BB * DIN).transpose(2, 0, 1, 3)
    w0, b0 = _prep_w(W_g0, b_g0, W_c0, b_c0, DIN)
    w1, b1 = _prep_w(W_g1, b_g1, W_c1, b_c1, U)

    full = lambda shape: pl.BlockSpec(shape, lambda b: (0,) * len(shape))
    hid, h2 = pl.pallas_call(
        _dcgru_body,
        grid=(B // BB,),
        in_specs=[
            pl.BlockSpec((1, M, N, BB * DIN), lambda b: (b, 0, 0, 0)),
            full((N, N)),
            full((N, N)),
            full((N, N)),
            full((N, N)),
            full((M * 2 * DIN, 4 * U)),
            full((1, 4 * U)),
            full((M * 2 * U, 4 * U)),
            full((1, 4 * U)),
        ],
        out_specs=[
            pl.BlockSpec((2, BB, N, U), lambda b: (0, b, 0, 0)),
            pl.BlockSpec((BB, N, U), lambda b: (b, 0, 0)),
        ],
        out_shape=[jax.ShapeDtypeStruct((2, B, N, U), f32),
                   jax.ShapeDtypeStruct((B, N, U), f32)],
        compiler_params=pltpu.CompilerParams(
            dimension_semantics=("parallel",)),
    )(tms, s1, s2, s1sq, s2sq, w0, b0, w1, b1)

    return (h2.reshape(B, N * U), hid.reshape(2, B, N * U))
